# baseline (speedup 1.0000x reference)
import functools

import jax
import jax.numpy as jnp
from jax import lax
from jax.experimental import pallas as pl
from jax.experimental.pallas import tpu as pltpu
from jax.experimental.pallas import tpu_sc as plsc

_MAX_STEPS = 1000
_N_ROWS = _MAX_STEPS + 1
_PAD_ROWS = 1008
_D_IN = 128
_D_OUT = 512


def _build_table():
    steps = jnp.arange(_N_ROWS, dtype=jnp.float32)[:, None]
    dims = jnp.arange(64, dtype=jnp.float32)[None, :]
    t = steps * (10.0 ** (dims * 4.0 / 63.0))
    table = jnp.stack([jnp.cos(t), jnp.sin(t)], axis=-1).reshape(_N_ROWS, -1)
    return jnp.pad(table, ((0, _PAD_ROWS - _N_ROWS), (0, 0)))


def _mlp_body(tab_ref, w1t_ref, b1_ref, w2t_ref, b2_ref, out_ref):
    h = jnp.dot(tab_ref[...], w1t_ref[...],
                preferred_element_type=jnp.float32)
    h = h + b1_ref[...]
    h = h * jax.nn.sigmoid(h)
    z = jnp.dot(h, w2t_ref[...],
                preferred_element_type=jnp.float32)
    z = z + b2_ref[...]
    out_ref[...] = z * jax.nn.sigmoid(z)


def _activated_table(table, W1, b1, W2, b2):
    return pl.pallas_call(
        _mlp_body,
        out_shape=jax.ShapeDtypeStruct((_PAD_ROWS, _D_OUT), jnp.float32),
    )(table, W1.T, b1.reshape(1, -1), W2.T, b2.reshape(1, -1))


@functools.cache
def _make_gather(batch, d):
    info = plsc.get_sparse_core_info()
    nc, ns, nl = info.num_cores, info.num_subcores, info.num_lanes
    nw = nc * ns
    b_per_w = batch // nw
    ch = 32
    n_ch = b_per_w // ch
    nbuf = 3
    nslab = 3
    mesh = plsc.VectorSubcoreMesh(core_axis_name="c", subcore_axis_name="s")

    @functools.partial(
        pl.kernel,
        mesh=mesh,
        out_type=jax.ShapeDtypeStruct((batch, d), jnp.float32),
        scratch_types=[
            pltpu.VMEM((b_per_w,), jnp.int32),
            pltpu.VMEM_SHARED((ns, nslab, ch, d), jnp.float32),
        ]
        + [pltpu.VMEM((ch, d), jnp.float32)] * nbuf
        + [pltpu.SemaphoreType.DMA] * (nbuf + 2 * nslab),
    )
    def gather_rows(tab_hbm, idx_hbm, out_hbm, idx_v, slab_sh, *bufs_sems):
        bufs = bufs_sems[:nbuf]
        gsems = bufs_sems[nbuf:2 * nbuf]
        xsems = bufs_sems[2 * nbuf:2 * nbuf + nslab]
        dsems = bufs_sems[2 * nbuf + nslab:]
        sid = lax.axis_index("s")
        wid = sid * nc + lax.axis_index("c")
        base = wid * b_per_w
        pltpu.sync_copy(idx_hbm.at[pl.ds(base, b_per_w)], idx_v)
        hi = jnp.full((nl,), _MAX_STEPS, dtype=jnp.int32)
        lo = jnp.zeros((nl,), dtype=jnp.int32)
        for i in range(b_per_w // nl):
            sl = pl.ds(i * nl, nl)
            idx_v[sl] = jnp.minimum(jnp.maximum(idx_v[sl], lo), hi)

        def start_gather(c):
            return pltpu.async_copy(
                tab_hbm.at[idx_v.at[pl.ds(c * ch, ch)]],
                bufs[c % nbuf], gsems[c % nbuf])

        def start_xfer(c):
            return pltpu.async_copy(
                bufs[c % nbuf], slab_sh.at[sid, c % nslab], xsems[c % nslab])

        def start_drain(c):
            return pltpu.async_copy(
                slab_sh.at[sid, c % nslab],
                out_hbm.at[pl.ds(base + c * ch, ch)], dsems[c % nslab])

        g = {c: start_gather(c) for c in range(min(nbuf, n_ch))}
        dr = {}
        for c in range(n_ch):
            g.pop(c).wait()
            if c - nslab >= 0:
                dr.pop(c - nslab).wait()
            start_xfer(c).wait()
            dr[c] = start_drain(c)
            if c + nbuf < n_ch:
                g[c + nbuf] = start_gather(c + nbuf)
        for c, h in dr.items():
            h.wait()

    return gather_rows


def kernel(diffusion_step, W1, b1, W2, b2):
    table = _build_table()
    z = _activated_table(table, W1, b1, W2, b2)
    return _make_gather(diffusion_step.shape[0], _D_OUT)(z, diffusion_step)


# speedup vs baseline: 1.0105x; 1.0105x over previous
import functools

import jax
import jax.numpy as jnp
from jax import lax
from jax.experimental import pallas as pl
from jax.experimental.pallas import tpu as pltpu
from jax.experimental.pallas import tpu_sc as plsc

_MAX_STEPS = 1000
_N_ROWS = _MAX_STEPS + 1
_PAD_ROWS = 1008
_D_IN = 128
_D_OUT = 512


def _build_table():
    steps = jnp.arange(_N_ROWS, dtype=jnp.float32)[:, None]
    dims = jnp.arange(64, dtype=jnp.float32)[None, :]
    t = steps * (10.0 ** (dims * 4.0 / 63.0))
    table = jnp.stack([jnp.cos(t), jnp.sin(t)], axis=-1).reshape(_N_ROWS, -1)
    return jnp.pad(table, ((0, _PAD_ROWS - _N_ROWS), (0, 0)))


def _mlp_body(tab_ref, w1t_ref, b1_ref, w2t_ref, b2_ref, out_ref):
    h = jnp.dot(tab_ref[...], w1t_ref[...],
                preferred_element_type=jnp.float32)
    h = h + b1_ref[...]
    h = h * jax.nn.sigmoid(h)
    z = jnp.dot(h, w2t_ref[...],
                preferred_element_type=jnp.float32)
    z = z + b2_ref[...]
    out_ref[...] = z * jax.nn.sigmoid(z)


def _activated_table(table, W1, b1, W2, b2):
    return pl.pallas_call(
        _mlp_body,
        out_shape=jax.ShapeDtypeStruct((_PAD_ROWS, _D_OUT), jnp.float32),
    )(table, W1.T, b1.reshape(1, -1), W2.T, b2.reshape(1, -1))


@functools.cache
def _make_gather(batch, d):
    info = plsc.get_sparse_core_info()
    nc, ns, nl = info.num_cores, info.num_subcores, info.num_lanes
    nw = nc * ns
    b_per_w = batch // nw
    ch = 64
    n_ch = b_per_w // ch
    nbuf = 3
    mesh = plsc.VectorSubcoreMesh(core_axis_name="c", subcore_axis_name="s")

    @functools.partial(
        pl.kernel,
        mesh=mesh,
        out_type=jax.ShapeDtypeStruct((batch, d), jnp.float32),
        compiler_params=pltpu.CompilerParams(use_tc_tiling_on_sc=True),
        scratch_types=[
            pltpu.VMEM((b_per_w,), jnp.int32),
        ]
        + [pltpu.VMEM((ch, d), jnp.float32)] * nbuf
        + [pltpu.SemaphoreType.DMA] * (2 * nbuf),
    )
    def gather_rows(tab_hbm, idx_hbm, out_hbm, idx_v, *bufs_sems):
        bufs = bufs_sems[:nbuf]
        gsems = bufs_sems[nbuf:2 * nbuf]
        ssems = bufs_sems[2 * nbuf:]
        wid = lax.axis_index("s") * nc + lax.axis_index("c")
        base = wid * b_per_w
        pltpu.sync_copy(idx_hbm.at[pl.ds(base, b_per_w)], idx_v)
        hi = jnp.full((nl,), _MAX_STEPS, dtype=jnp.int32)
        lo = jnp.zeros((nl,), dtype=jnp.int32)
        for i in range(b_per_w // nl):
            sl = pl.ds(i * nl, nl)
            idx_v[sl] = jnp.minimum(jnp.maximum(idx_v[sl], lo), hi)

        def start_gather(c):
            return pltpu.async_copy(
                tab_hbm.at[idx_v.at[pl.ds(c * ch, ch)]],
                bufs[c % nbuf], gsems[c % nbuf])

        def start_store(c):
            return pltpu.async_copy(
                bufs[c % nbuf], out_hbm.at[pl.ds(base + c * ch, ch)],
                ssems[c % nbuf])

        g = {c: start_gather(c) for c in range(min(2, n_ch))}
        st = {}
        for c in range(n_ch):
            g.pop(c).wait()
            st[c] = start_store(c)
            if c + 2 < n_ch:
                if c - 1 >= 0:
                    st.pop(c - 1).wait()
                g[c + 2] = start_gather(c + 2)
        for c, h in st.items():
            h.wait()

    return gather_rows


def kernel(diffusion_step, W1, b1, W2, b2):
    table = _build_table()
    z = _activated_table(table, W1, b1, W2, b2)
    return _make_gather(diffusion_step.shape[0], _D_OUT)(z, diffusion_step)


# speedup vs baseline: 1.0795x; 1.0682x over previous
import functools

import jax
import jax.numpy as jnp
from jax import lax
from jax.experimental import pallas as pl
from jax.experimental.pallas import tpu as pltpu
from jax.experimental.pallas import tpu_sc as plsc

_MAX_STEPS = 1000
_N_ROWS = _MAX_STEPS + 1
_PAD_ROWS = 1008
_D_IN = 128
_D_OUT = 512


def _build_table():
    steps = jnp.arange(_N_ROWS, dtype=jnp.float32)[:, None]
    dims = jnp.arange(64, dtype=jnp.float32)[None, :]
    t = steps * (10.0 ** (dims * 4.0 / 63.0))
    table = jnp.stack([jnp.cos(t), jnp.sin(t)], axis=-1).reshape(_N_ROWS, -1)
    return jnp.pad(table, ((0, _PAD_ROWS - _N_ROWS), (0, 0)))


_DN = (((1,), (1,)), ((), ()))


def _mlp_body(tab_ref, w1_ref, b1_ref, w2_ref, b2_ref, out_ref):
    h = lax.dot_general(tab_ref[...], w1_ref[...], _DN,
                        preferred_element_type=jnp.float32)
    h = h + b1_ref[...]
    h = h * jax.nn.sigmoid(h)
    z = lax.dot_general(h, w2_ref[...], _DN,
                        preferred_element_type=jnp.float32)
    z = z + b2_ref[...]
    out_ref[...] = z * jax.nn.sigmoid(z)


def _activated_table(table, W1, b1, W2, b2):
    return pl.pallas_call(
        _mlp_body,
        out_shape=jax.ShapeDtypeStruct((_PAD_ROWS, _D_OUT), jnp.float32),
    )(table, W1, b1.reshape(1, -1), W2, b2.reshape(1, -1))


@functools.cache
def _make_gather(batch, d):
    info = plsc.get_sparse_core_info()
    nc, ns = info.num_cores, info.num_subcores
    nw = nc * ns
    b_per_w = batch // nw
    ch = 64
    n_ch = b_per_w // ch
    nbuf = 2
    n_grp = n_ch // nbuf
    mesh = plsc.VectorSubcoreMesh(core_axis_name="c", subcore_axis_name="s")

    @functools.partial(
        pl.kernel,
        mesh=mesh,
        out_type=jax.ShapeDtypeStruct((batch, d), jnp.float32),
        scratch_types=[
            pltpu.VMEM((b_per_w,), jnp.int32),
        ]
        + [pltpu.VMEM((ch, d), jnp.float32)] * nbuf
        + [pltpu.SemaphoreType.DMA] * (2 * nbuf),
    )
    def gather_rows(tab_hbm, idx_hbm, out_hbm, idx_v, *bufs_sems):
        bufs = bufs_sems[:nbuf]
        gsems = bufs_sems[nbuf:2 * nbuf]
        ssems = bufs_sems[2 * nbuf:]
        wid = lax.axis_index("s") * nc + lax.axis_index("c")
        base = wid * b_per_w
        pltpu.sync_copy(idx_hbm.at[pl.ds(base, b_per_w)], idx_v)

        def gather_cp(c, b):
            return pltpu.make_async_copy(
                tab_hbm.at[idx_v.at[pl.ds(c * ch, ch)]], bufs[b], gsems[b])

        def store_cp(c, b):
            return pltpu.make_async_copy(
                bufs[b], out_hbm.at[pl.ds(base + c * ch, ch)], ssems[b])

        for b in range(nbuf):
            gather_cp(b, b).start()

        @pl.loop(0, n_grp)
        def _(grp):
            for b in range(nbuf):
                c = grp * nbuf + b
                gather_cp(c, b).wait()
                store_cp(c, b).start()

                @pl.when(grp < n_grp - 1)
                def _():
                    store_cp(c, b).wait()
                    gather_cp(c + nbuf, b).start()

        for b in range(nbuf):
            store_cp(n_ch - nbuf + b, b).wait()

    return gather_rows


def kernel(diffusion_step, W1, b1, W2, b2):
    table = _build_table()
    z = _activated_table(table, W1, b1, W2, b2)
    idx = jnp.clip(diffusion_step, 0, _MAX_STEPS).astype(jnp.int32)
    return _make_gather(diffusion_step.shape[0], _D_OUT)(z, idx)
